# baseline (device time: 66735 ns/iter reference)
import jax
import jax.numpy as jnp
from jax import lax
from jax.experimental import pallas as pl
from jax.experimental.pallas import tpu as pltpu

N_DEV = 4
G = 8


def kernel(x):
    m, n = x.shape
    c = m // G

    def body(x_hbm, out_ref, comm_ref, recv_ref, copy_sem, send_sems, recv_sems):
        my = lax.axis_index("i")

        cp = pltpu.make_async_copy(x_hbm, out_ref, copy_sem)
        cp.start()

        barrier_sem = pltpu.get_barrier_semaphore()
        for j in range(N_DEV):
            @pl.when(j != my)
            def _():
                pl.semaphore_signal(
                    barrier_sem, inc=1,
                    device_id=(j,), device_id_type=pl.DeviceIdType.MESH,
                )
        pl.semaphore_wait(barrier_sem, N_DEV - 1)
        cp.wait()

        y = out_ref[:, :].reshape(c, G, n)
        s = 1
        while s < G:
            shifted = jnp.concatenate(
                [jnp.ones((c, s, n), jnp.float32), y[:, : G - s, :]], axis=1
            )
            y = y * shifted
            s *= 2

        t = y[:, G - 1, :]
        s = 1
        while s < c:
            t = t * jnp.concatenate(
                [jnp.ones((s, n), jnp.float32), t[: c - s, :]], axis=0
            )
            s *= 2

        comm_ref[0, :] = t[c - 1, :]
        for j in range(N_DEV):
            @pl.when(j != my)
            def _():
                pltpu.make_async_remote_copy(
                    src_ref=comm_ref,
                    dst_ref=recv_ref.at[my],
                    send_sem=send_sems.at[j],
                    recv_sem=recv_sems.at[my],
                    device_id=(j,),
                    device_id_type=pl.DeviceIdType.MESH,
                ).start()

        excl = jnp.concatenate(
            [jnp.ones((1, n), jnp.float32), t[: c - 1, :]], axis=0
        )

        acc = jnp.ones((1, n), jnp.float32)
        for j in range(N_DEV):
            @pl.when(j != my)
            def _():
                w = pltpu.make_async_remote_copy(
                    src_ref=comm_ref,
                    dst_ref=recv_ref.at[j],
                    send_sem=send_sems.at[j],
                    recv_sem=recv_sems.at[j],
                    device_id=(j,),
                    device_id_type=pl.DeviceIdType.MESH,
                )
                w.wait_recv()
                w.wait_send()
            acc = acc * jnp.where(j < my, recv_ref[j, :, :], jnp.float32(1.0))

        scale = excl * acc
        out_ref[:, :] = (y * scale[:, None, :]).reshape(m, n)

    return pl.pallas_call(
        body,
        out_shape=jax.ShapeDtypeStruct((m, n), jnp.float32),
        in_specs=[pl.BlockSpec(memory_space=pl.ANY)],
        out_specs=pl.BlockSpec(memory_space=pltpu.VMEM),
        scratch_shapes=[
            pltpu.VMEM((1, n), jnp.float32),
            pltpu.VMEM((N_DEV, 1, n), jnp.float32),
            pltpu.SemaphoreType.DMA,
            pltpu.SemaphoreType.DMA((N_DEV,)),
            pltpu.SemaphoreType.DMA((N_DEV,)),
        ],
        compiler_params=pltpu.CompilerParams(
            collective_id=0,
            vmem_limit_bytes=100 * 1024 * 1024,
        ),
    )(x)


# device time: 46201 ns/iter; 1.4444x vs baseline; 1.4444x over previous
import jax
import jax.numpy as jnp
from jax import lax
from jax.experimental import pallas as pl
from jax.experimental.pallas import tpu as pltpu

N_DEV = 4


def kernel(x):
    m, n = x.shape
    n_steps = (m - 1).bit_length()
    assert n_steps % 2 == 0, "step parity assumed below"

    def body(x_ref, out_ref, ping_ref, comm_ref, recv_ref, send_sems, recv_sems):
        my = lax.axis_index("i")

        barrier_sem = pltpu.get_barrier_semaphore()
        for j in range(N_DEV):
            @pl.when(j != my)
            def _():
                pl.semaphore_signal(
                    barrier_sem, inc=1,
                    device_id=(j,), device_id_type=pl.DeviceIdType.MESH,
                )
        pl.semaphore_wait(barrier_sem, N_DEV - 1)

        half = m // 2
        tot = x_ref[pl.ds(0, half), :] * x_ref[pl.ds(half, half), :]
        while half > 1:
            half //= 2
            tot = tot[:half, :] * tot[half:, :]
        comm_ref[0, :] = tot[0, :]
        for j in range(N_DEV):
            @pl.when(j != my)
            def _():
                pltpu.make_async_remote_copy(
                    src_ref=comm_ref,
                    dst_ref=recv_ref.at[my],
                    send_sem=send_sems.at[j],
                    recv_sem=recv_sems.at[my],
                    device_id=(j,),
                    device_id_type=pl.DeviceIdType.MESH,
                ).start()

        acc = None
        for k in range(1, n_steps + 1):
            s = 1 << (k - 1)
            src = x_ref if k == 1 else (ping_ref if k % 2 == 0 else out_ref)
            dst = ping_ref if k % 2 == 1 else out_ref
            if k < n_steps:
                dst[pl.ds(0, s), :] = src[pl.ds(0, s), :]
                dst[pl.ds(s, m - s), :] = (
                    src[pl.ds(s, m - s), :] * src[pl.ds(0, m - s), :]
                )
            else:
                acc = jnp.ones((1, n), jnp.float32)
                for j in range(N_DEV):
                    @pl.when(j != my)
                    def _():
                        w = pltpu.make_async_remote_copy(
                            src_ref=comm_ref,
                            dst_ref=recv_ref.at[j],
                            send_sem=send_sems.at[j],
                            recv_sem=recv_sems.at[j],
                            device_id=(j,),
                            device_id_type=pl.DeviceIdType.MESH,
                        )
                        w.wait_recv()
                        w.wait_send()
                    acc = acc * jnp.where(
                        j < my, recv_ref[j, :, :], jnp.float32(1.0)
                    )
                dst[pl.ds(0, s), :] = src[pl.ds(0, s), :] * acc
                dst[pl.ds(s, m - s), :] = (
                    src[pl.ds(s, m - s), :] * src[pl.ds(0, m - s), :] * acc
                )

    return pl.pallas_call(
        body,
        out_shape=jax.ShapeDtypeStruct((m, n), jnp.float32),
        in_specs=[pl.BlockSpec(memory_space=pltpu.VMEM)],
        out_specs=pl.BlockSpec(memory_space=pltpu.VMEM),
        scratch_shapes=[
            pltpu.VMEM((m, n), jnp.float32),
            pltpu.VMEM((1, n), jnp.float32),
            pltpu.VMEM((N_DEV, 1, n), jnp.float32),
            pltpu.SemaphoreType.DMA((N_DEV,)),
            pltpu.SemaphoreType.DMA((N_DEV,)),
        ],
        compiler_params=pltpu.CompilerParams(
            collective_id=0,
            vmem_limit_bytes=100 * 1024 * 1024,
        ),
    )(x)


# device time: 46186 ns/iter; 1.4449x vs baseline; 1.0003x over previous
import jax
import jax.numpy as jnp
from jax import lax
from jax.experimental import pallas as pl
from jax.experimental.pallas import tpu as pltpu

N_DEV = 4


def kernel(x):
    m, n = x.shape
    n_steps = (m - 1).bit_length()

    def body(x_hbm, out_ref, ping_ref, comm_ref, recv_ref, copy_sem,
             send_sems, recv_sems):
        my = lax.axis_index("i")

        cp = pltpu.make_async_copy(x_hbm, out_ref, copy_sem)
        cp.start()

        barrier_sem = pltpu.get_barrier_semaphore()
        for j in range(N_DEV):
            @pl.when(j != my)
            def _():
                pl.semaphore_signal(
                    barrier_sem, inc=1,
                    device_id=(j,), device_id_type=pl.DeviceIdType.MESH,
                )
        pl.semaphore_wait(barrier_sem, N_DEV - 1)
        cp.wait()

        bufs = (out_ref, ping_ref)
        for k in range(n_steps):
            if k == n_steps - 1:
                src = bufs[k % 2]
                comm_ref[0, :] = (
                    src[pl.ds(m // 2 - 1, 1), :] * src[pl.ds(m - 1, 1), :]
                )[0, :]
                for j in range(N_DEV):
                    @pl.when(j != my)
                    def _():
                        pltpu.make_async_remote_copy(
                            src_ref=comm_ref,
                            dst_ref=recv_ref.at[my],
                            send_sem=send_sems.at[j],
                            recv_sem=recv_sems.at[my],
                            device_id=(j,),
                            device_id_type=pl.DeviceIdType.MESH,
                        ).start()
            s = 1 << k
            src = bufs[k % 2]
            dst = bufs[(k + 1) % 2]
            dst[pl.ds(0, s), :] = src[pl.ds(0, s), :]
            dst[pl.ds(s, m - s), :] = (
                src[pl.ds(s, m - s), :] * src[pl.ds(0, m - s), :]
            )

        acc = jnp.ones((1, n), jnp.float32)
        for j in range(N_DEV):
            @pl.when(j != my)
            def _():
                w = pltpu.make_async_remote_copy(
                    src_ref=comm_ref,
                    dst_ref=recv_ref.at[j],
                    send_sem=send_sems.at[j],
                    recv_sem=recv_sems.at[j],
                    device_id=(j,),
                    device_id_type=pl.DeviceIdType.MESH,
                )
                w.wait_recv()
                w.wait_send()
            acc = acc * jnp.where(j < my, recv_ref[j, :, :], jnp.float32(1.0))

        out_ref[:, :] = out_ref[:, :] * acc

    return pl.pallas_call(
        body,
        out_shape=jax.ShapeDtypeStruct((m, n), jnp.float32),
        in_specs=[pl.BlockSpec(memory_space=pl.ANY)],
        out_specs=pl.BlockSpec(memory_space=pltpu.VMEM),
        scratch_shapes=[
            pltpu.VMEM((m, n), jnp.float32),
            pltpu.VMEM((1, n), jnp.float32),
            pltpu.VMEM((N_DEV, 1, n), jnp.float32),
            pltpu.SemaphoreType.DMA,
            pltpu.SemaphoreType.DMA((N_DEV,)),
            pltpu.SemaphoreType.DMA((N_DEV,)),
        ],
        compiler_params=pltpu.CompilerParams(
            collective_id=0,
            vmem_limit_bytes=100 * 1024 * 1024,
        ),
    )(x)


# device time: 46061 ns/iter; 1.4488x vs baseline; 1.0027x over previous
import jax
import jax.numpy as jnp
from jax import lax
from jax.experimental import pallas as pl
from jax.experimental.pallas import tpu as pltpu

N_DEV = 4
P = 64


def kernel(x):
    m, n = x.shape
    r = m // P
    w = P * n

    def body(x_hbm, out_ref, z1, z2, comm_ref, copy_sem, send_sems, recv_sems):
        my = lax.axis_index("i")
        left = (my - 1) % N_DEV
        right = (my + 1) % N_DEV

        barrier_sem = pltpu.get_barrier_semaphore()
        for nbr in (left, right):
            pl.semaphore_signal(
                barrier_sem, inc=1,
                device_id=(nbr,), device_id_type=pl.DeviceIdType.MESH,
            )
        pl.semaphore_wait(barrier_sem, 2)

        cp = pltpu.make_async_copy(x_hbm, out_ref, copy_sem)
        cp.start()
        cp.wait()

        for p in range(P):
            z1[pl.ds(0, 1), pl.ds(p * n, n)] = out_ref[pl.ds(p * r, 1), :]
            z1[pl.ds(1, r - 1), pl.ds(p * n, n)] = (
                out_ref[pl.ds(p * r + 1, r - 1), :]
                * out_ref[pl.ds(p * r, r - 1), :]
            )

        cur, nxt = z1, z2
        s = 2
        while s < r:
            nxt[pl.ds(0, s), :] = cur[pl.ds(0, s), :]
            nxt[pl.ds(s, r - s), :] = (
                cur[pl.ds(s, r - s), :] * cur[pl.ds(0, r - s), :]
            )
            cur, nxt = nxt, cur
            s *= 2

        t = cur[pl.ds(r - 1, 1), :]
        sb = 1
        while sb < P:
            t = t * jnp.concatenate(
                [jnp.ones((1, sb * n), jnp.float32), t[:, : w - sb * n]],
                axis=1,
            )
            sb *= 2
        excl = jnp.concatenate(
            [jnp.ones((1, n), jnp.float32), t[:, : w - n]], axis=1
        )
        comm_ref[0, :, :] = t[:, w - n:]

        acc = jnp.ones((1, n), jnp.float32)
        for h in range(N_DEV - 1):
            send_slot = h % 2
            recv_slot = (h + 1) % 2
            rdma = pltpu.make_async_remote_copy(
                src_ref=comm_ref.at[send_slot],
                dst_ref=comm_ref.at[recv_slot],
                send_sem=send_sems.at[send_slot],
                recv_sem=recv_sems.at[recv_slot],
                device_id=(right,),
                device_id_type=pl.DeviceIdType.MESH,
            )
            rdma.start()
            rdma.wait()
            origin = (my - h - 1) % N_DEV
            chunk = comm_ref[recv_slot, :, :]
            acc = acc * jnp.where(origin < my, chunk, jnp.float32(1.0))

        for p in range(P):
            out_ref[pl.ds(p * r, r), :] = cur[:, pl.ds(p * n, n)] * (
                excl[:, p * n: (p + 1) * n] * acc
            )

    return pl.pallas_call(
        body,
        out_shape=jax.ShapeDtypeStruct((m, n), jnp.float32),
        in_specs=[pl.BlockSpec(memory_space=pl.ANY)],
        out_specs=pl.BlockSpec(memory_space=pltpu.VMEM),
        scratch_shapes=[
            pltpu.VMEM((m // P, P * n), jnp.float32),
            pltpu.VMEM((m // P, P * n), jnp.float32),
            pltpu.VMEM((2, 1, n), jnp.float32),
            pltpu.SemaphoreType.DMA,
            pltpu.SemaphoreType.DMA((2,)),
            pltpu.SemaphoreType.DMA((2,)),
        ],
        compiler_params=pltpu.CompilerParams(
            collective_id=0,
            vmem_limit_bytes=100 * 1024 * 1024,
        ),
    )(x)


# device time: 37115 ns/iter; 1.7981x vs baseline; 1.2410x over previous
import jax
import jax.numpy as jnp
from jax import lax
from jax.experimental import pallas as pl
from jax.experimental.pallas import tpu as pltpu

N_DEV = 4


def kernel(x):
    m, n = x.shape

    def body(x_hbm, out_ref, ping_ref, comm_ref, copy_sem, send_sems, recv_sems):
        my = lax.axis_index("i")
        left = (my - 1) % N_DEV
        right = (my + 1) % N_DEV

        barrier_sem = pltpu.get_barrier_semaphore()
        for nbr in (left, right):
            pl.semaphore_signal(
                barrier_sem, inc=1,
                device_id=(nbr,), device_id_type=pl.DeviceIdType.MESH,
            )
        pl.semaphore_wait(barrier_sem, 2)

        cp = pltpu.make_async_copy(x_hbm, out_ref, copy_sem)
        cp.start()
        cp.wait()

        bufs = (out_ref, ping_ref)
        s, k = 1, 0
        while s < m:
            src = bufs[k % 2]
            dst = bufs[(k + 1) % 2]
            dst[pl.ds(0, s), :] = src[pl.ds(0, s), :]
            dst[pl.ds(s, m - s), :] = (
                src[pl.ds(s, m - s), :] * src[pl.ds(0, m - s), :]
            )
            s *= 2
            k += 1
        assert k % 2 == 0, "odd number of scan steps: result in ping_ref"

        comm_ref[0, :, :] = out_ref[pl.ds(m - 1, 1), :]

        acc = jnp.ones((1, n), jnp.float32)
        for h in range(N_DEV - 1):
            send_slot = h % 2
            recv_slot = (h + 1) % 2
            rdma = pltpu.make_async_remote_copy(
                src_ref=comm_ref.at[send_slot],
                dst_ref=comm_ref.at[recv_slot],
                send_sem=send_sems.at[send_slot],
                recv_sem=recv_sems.at[recv_slot],
                device_id=(right,),
                device_id_type=pl.DeviceIdType.MESH,
            )
            rdma.start()
            rdma.wait()
            origin = (my - h - 1) % N_DEV
            chunk = comm_ref[recv_slot, :, :]
            acc = acc * jnp.where(origin < my, chunk, jnp.float32(1.0))

        out_ref[:, :] = out_ref[:, :] * acc

    return pl.pallas_call(
        body,
        out_shape=jax.ShapeDtypeStruct((m, n), jnp.float32),
        in_specs=[pl.BlockSpec(memory_space=pl.ANY)],
        out_specs=pl.BlockSpec(memory_space=pltpu.VMEM),
        scratch_shapes=[
            pltpu.VMEM((m, n), jnp.float32),
            pltpu.VMEM((2, 1, n), jnp.float32),
            pltpu.SemaphoreType.DMA,
            pltpu.SemaphoreType.DMA((2,)),
            pltpu.SemaphoreType.DMA((2,)),
        ],
        compiler_params=pltpu.CompilerParams(collective_id=0),
    )(x)
